# baseline (speedup 1.0000x reference)
import functools

import jax
import jax.numpy as jnp
from jax import lax
from jax.experimental import pallas as pl
from jax.experimental.pallas import tpu as pltpu
from jax.experimental.pallas import tpu_sc as plsc

_NUM_FIELDS = 26
_VOCAB = 100000
_EMB = 32
_BATCH = 16384

_NC = 2
_NS = 16
_NW = _NC * _NS
_L = 16
_FP = 4096
_NP = _BATCH // _FP
_H0 = 50048
_H1 = _VOCAB - _H0


def _body(feats_hbm, table_hbm, out_hbm, row0_v, row1_v, feats_v, acc_v,
          rsem0, rsem1, fsem0, fsem1):
    e = lax.axis_index("s") * _NC + lax.axis_index("c")
    zeros = jnp.zeros((_L,), jnp.float32)

    @pl.loop(0, _BATCH // _L, unroll=8)
    def _zb(i):
        acc_v[pl.ds(i * _L, _L)] = zeros

    rows = (row0_v, row1_v)
    rsems = (rsem0, rsem1)
    fsems = (fsem0, fsem1)

    def row_src(f, h):
        if h == 0:
            return table_hbm.at[f, e, pl.ds(0, _H0)]
        return table_hbm.at[f, e, pl.ds(_H0, _H1)]

    def start_row(f, h):
        pltpu.async_copy(row_src(f, h), rows[h], rsems[h])

    def wait_row(f, h):
        pltpu.make_async_copy(row_src(f, h), rows[h], rsems[h]).wait()

    def feats_src(f, q):
        return feats_hbm.at[f, pl.ds(q * _FP, _FP)]

    def start_feats(f, q, p):
        pltpu.async_copy(feats_src(f, q), feats_v.at[p], fsems[p])

    def wait_feats(f, q, p):
        pltpu.make_async_copy(feats_src(f, q), feats_v.at[p], fsems[p]).wait()

    def gather_pass(f, h):
        start_feats(f, 0, 0)
        for q0 in range(0, _NP, 2):
            for p in range(2):
                q = q0 + p
                wait_feats(f, q, p)
                if q < _NP - 1:
                    start_feats(f, q + 1, (p + 1) % 2)
                base = q * _FP

                @pl.loop(0, _FP // _L, unroll=8)
                def _jb(j):
                    idx = feats_v[p, pl.ds(j * _L, _L)]
                    if h == 0:
                        valid = idx < _H0
                        loc = jnp.minimum(idx, _H0 - 1)
                        v = plsc.load_gather(row0_v, [loc])
                    else:
                        valid = idx >= _H0
                        loc = jnp.maximum(idx - _H0, 0)
                        v = plsc.load_gather(row1_v, [loc])
                    v = jnp.where(valid, v, 0.0)
                    plsc.addupdate(acc_v.at[pl.ds(base + j * _L, _L)], v)

    start_row(jnp.int32(0), 0)

    @pl.loop(0, _NUM_FIELDS)
    def _main(f):
        wait_row(f, 0)
        start_row(f, 1)
        gather_pass(f, 0)
        wait_row(f, 1)

        @pl.when(f < _NUM_FIELDS - 1)
        def _pref():
            start_row(f + 1, 0)

        gather_pass(f, 1)

    pltpu.sync_copy(acc_v, out_hbm.at[e])


_embed_sum = functools.partial(
    pl.kernel,
    out_type=jax.ShapeDtypeStruct((_EMB, _BATCH), jnp.float32),
    mesh=plsc.VectorSubcoreMesh(core_axis_name="c", subcore_axis_name="s"),
    compiler_params=pltpu.CompilerParams(needs_layout_passes=False),
    scratch_types=[
        pltpu.VMEM((_H0,), jnp.float32),
        pltpu.VMEM((_H1,), jnp.float32),
        pltpu.VMEM((2, _FP), jnp.int32),
        pltpu.VMEM((_BATCH,), jnp.float32),
        pltpu.SemaphoreType.DMA,
        pltpu.SemaphoreType.DMA,
        pltpu.SemaphoreType.DMA,
        pltpu.SemaphoreType.DMA,
    ],
)(_body)


def kernel(categorical_feats, tables):
    feats_t = categorical_feats.astype(jnp.int32).T
    tables_t = tables.transpose(0, 2, 1)
    return _embed_sum(feats_t, tables_t).T
